# baseline (speedup 1.0000x reference)
import functools

import jax
import jax.numpy as jnp
from jax import lax
from jax.experimental import pallas as pl
from jax.experimental.pallas import tpu as pltpu
from jax.experimental.pallas import tpu_sc as plsc

_LANES = 16
_SC_ROWS_PER_WORKER = 12


def _make_sc_add(batch, maxlen, dim):
    info = plsc.get_sparse_core_info()
    nc, ns = info.num_cores, info.num_subcores
    nw = nc * ns
    assert batch % nw == 0 and dim % _LANES == 0
    b_per_w = batch // nw
    d_chunks = dim // _LANES

    mesh = plsc.VectorSubcoreMesh(core_axis_name="c", subcore_axis_name="s")

    @functools.partial(
        pl.kernel,
        out_type=jax.ShapeDtypeStruct((batch, maxlen, dim), jnp.float32),
        mesh=mesh,
        scratch_types=[
            pltpu.VMEM((maxlen, dim), jnp.float32),
            pltpu.VMEM((maxlen, dim), jnp.float32),
            pltpu.VMEM((maxlen, dim), jnp.float32),
            pltpu.VMEM((maxlen, dim), jnp.float32),
            pltpu.VMEM((maxlen, dim), jnp.float32),
            pltpu.SemaphoreType.DMA,
            pltpu.SemaphoreType.DMA,
            pltpu.SemaphoreType.DMA,
            pltpu.SemaphoreType.DMA,
        ],
    )
    def sc_add(x_hbm, pos_hbm, out_hbm, pos_v, ib0, ib1, ob0, ob1,
               is0, is1, os0, os1):
        wid = lax.axis_index("s") * nc + lax.axis_index("c")
        base = wid * b_per_w
        ibs, obs = [ib0, ib1], [ob0, ob1]
        isems, osems = [is0, is1], [os0, os1]

        pltpu.sync_copy(pos_hbm, pos_v)
        pltpu.async_copy(x_hbm.at[base], ibs[0], isems[0])
        pltpu.async_copy(x_hbm.at[base + 1], ibs[1], isems[1])

        for r in range(b_per_w):
            p = r % 2
            pltpu.make_async_copy(x_hbm.at[base + r], ibs[p], isems[p]).wait()
            if r >= 2:
                pltpu.make_async_copy(
                    obs[p], out_hbm.at[base + r - 2], osems[p]).wait()

            @plsc.parallel_loop(0, maxlen, unroll=2)
            def _add(i, _p=p):
                for j in range(d_chunks):
                    sl = pl.ds(j * _LANES, _LANES)
                    obs[_p][i, sl] = ibs[_p][i, sl] + pos_v[i, sl]

            pltpu.async_copy(obs[p], out_hbm.at[base + r], osems[p])
            if r + 2 < b_per_w:
                pltpu.async_copy(x_hbm.at[base + r + 2], ibs[p], isems[p])

        for r in (b_per_w - 2, b_per_w - 1):
            p = r % 2
            pltpu.make_async_copy(obs[p], out_hbm.at[base + r], osems[p]).wait()

    return sc_add


def _tc_add(x, pos, b_blk=32):
    batch, maxlen, dim = x.shape

    def body(x_ref, pos_ref, o_ref):
        o_ref[...] = x_ref[...] + pos_ref[...][None]

    return pl.pallas_call(
        body,
        grid=(batch // b_blk,),
        in_specs=[
            pl.BlockSpec((b_blk, maxlen, dim), lambda i: (i, 0, 0)),
            pl.BlockSpec((maxlen, dim), lambda i: (0, 0)),
        ],
        out_specs=pl.BlockSpec((b_blk, maxlen, dim), lambda i: (i, 0, 0)),
        out_shape=jax.ShapeDtypeStruct((batch, maxlen, dim), jnp.float32),
    )(x, pos)


def kernel(x, pos_emb):
    batch, maxlen, dim = x.shape
    batch_sc = 32 * _SC_ROWS_PER_WORKER
    batch_tc = batch - batch_sc
    x_tc, x_sc = x[:batch_tc], x[batch_tc:]
    out_tc = _tc_add(x_tc, pos_emb)
    out_sc = _make_sc_add(batch_sc, maxlen, dim)(x_sc, pos_emb)
    return jnp.concatenate([out_tc, out_sc], axis=0)


# speedup vs baseline: 1.4388x; 1.4388x over previous
import functools

import jax
import jax.numpy as jnp
from jax import lax
from jax.experimental import pallas as pl
from jax.experimental.pallas import tpu as pltpu
from jax.experimental.pallas import tpu_sc as plsc

_LANES = 16


def _make_sc_add(batch, row0, batch_sc, maxlen, dim):
    info = plsc.get_sparse_core_info()
    nc, ns = info.num_cores, info.num_subcores
    nw = nc * ns
    assert batch_sc % nw == 0 and dim % _LANES == 0
    b_per_w = batch_sc // nw
    d_chunks = dim // _LANES

    mesh = plsc.VectorSubcoreMesh(core_axis_name="c", subcore_axis_name="s")

    @functools.partial(
        pl.kernel,
        out_type=jax.ShapeDtypeStruct((batch_sc, maxlen, dim), jnp.float32),
        mesh=mesh,
        scratch_types=[
            pltpu.VMEM((maxlen, dim), jnp.float32),
            pltpu.VMEM((maxlen, dim), jnp.float32),
            pltpu.VMEM((maxlen, dim), jnp.float32),
            pltpu.VMEM((maxlen, dim), jnp.float32),
            pltpu.VMEM((maxlen, dim), jnp.float32),
            pltpu.SemaphoreType.DMA,
            pltpu.SemaphoreType.DMA,
            pltpu.SemaphoreType.DMA,
            pltpu.SemaphoreType.DMA,
        ],
    )
    def sc_add(x_hbm, pos_hbm, out_hbm, pos_v, ib0, ib1, ob0, ob1,
               is0, is1, os0, os1):
        wid = lax.axis_index("s") * nc + lax.axis_index("c")
        base = wid * b_per_w
        xbase = row0 + base
        ibs, obs = [ib0, ib1], [ob0, ob1]
        isems, osems = [is0, is1], [os0, os1]

        pltpu.sync_copy(pos_hbm, pos_v)
        pltpu.async_copy(x_hbm.at[xbase], ibs[0], isems[0])
        pltpu.async_copy(x_hbm.at[xbase + 1], ibs[1], isems[1])

        for r in range(b_per_w):
            p = r % 2
            pltpu.make_async_copy(x_hbm.at[xbase + r], ibs[p], isems[p]).wait()
            if r >= 2:
                pltpu.make_async_copy(
                    obs[p], out_hbm.at[base + r - 2], osems[p]).wait()

            @plsc.parallel_loop(0, maxlen, unroll=2)
            def _add(i, _p=p):
                for j in range(d_chunks):
                    sl = pl.ds(j * _LANES, _LANES)
                    obs[_p][i, sl] = ibs[_p][i, sl] + pos_v[i, sl]

            pltpu.async_copy(obs[p], out_hbm.at[base + r], osems[p])
            if r + 2 < b_per_w:
                pltpu.async_copy(x_hbm.at[xbase + r + 2], ibs[p], isems[p])

        for r in (b_per_w - 2, b_per_w - 1):
            p = r % 2
            pltpu.make_async_copy(obs[p], out_hbm.at[base + r], osems[p]).wait()

    return sc_add


def _tc_add(x, pos, batch_tc, b_blk=32):
    batch, maxlen, dim = x.shape

    def body(x_ref, pos_ref, o_ref):
        o_ref[...] = x_ref[...] + pos_ref[...][None]

    return pl.pallas_call(
        body,
        grid=(batch_tc // b_blk,),
        in_specs=[
            pl.BlockSpec((b_blk, maxlen, dim), lambda i: (i, 0, 0)),
            pl.BlockSpec((maxlen, dim), lambda i: (0, 0)),
        ],
        out_specs=pl.BlockSpec((b_blk, maxlen, dim), lambda i: (i, 0, 0)),
        out_shape=jax.ShapeDtypeStruct((batch_tc, maxlen, dim), jnp.float32),
    )(x, pos)


def kernel(x, pos_emb):
    batch, maxlen, dim = x.shape
    batch_sc = 384
    batch_tc = batch - batch_sc
    out_tc = _tc_add(x, pos_emb, batch_tc)
    out_sc = _make_sc_add(batch, batch_tc, batch_sc, maxlen, dim)(x, pos_emb)
    return jnp.concatenate([out_tc, out_sc], axis=0)


# speedup vs baseline: 2.0186x; 1.4030x over previous
import functools

import jax
import jax.numpy as jnp
from jax import lax
from jax.experimental import pallas as pl
from jax.experimental.pallas import tpu as pltpu
from jax.experimental.pallas import tpu_sc as plsc

_LANES = 16
_NSLOT = 2
_CH = (104, 96)


def _make_sc_add(batch, maxlen, dim):
    info = plsc.get_sparse_core_info()
    nc, ns = info.num_cores, info.num_subcores
    nw = nc * ns
    assert batch % nw == 0 and dim % _LANES == 0 and sum(_CH) == maxlen
    b_per_w = batch // nw
    sp_rows = ns * _NSLOT * maxlen

    mesh = plsc.VectorSubcoreMesh(core_axis_name="c", subcore_axis_name="s")

    @functools.partial(
        pl.kernel,
        out_type=jax.ShapeDtypeStruct((batch, maxlen, dim), jnp.float32),
        mesh=mesh,
        scratch_types=[
            pltpu.VMEM((maxlen, dim), jnp.float32),
            pltpu.VMEM((_NSLOT, _CH[0]), jnp.int32),
            pltpu.VMEM((_NSLOT, _CH[1]), jnp.int32),
            pltpu.VMEM_SHARED((sp_rows, dim), jnp.float32),
            pltpu.SemaphoreType.DMA,
            pltpu.SemaphoreType.DMA,
            pltpu.SemaphoreType.DMA,
            pltpu.SemaphoreType.DMA,
            pltpu.SemaphoreType.DMA,
            pltpu.SemaphoreType.DMA,
        ],
    )
    def sc_add(x_hbm, pos_hbm, out_hbm, pos_v, idxa, idxb, sp,
               in0, in1, ad0, ad1, ot0, ot1):
        cid = lax.axis_index("c")
        sid = lax.axis_index("s")
        wid = sid * nc + cid
        base = wid * b_per_w
        tile_base = sid * (_NSLOT * maxlen)
        insems, addsems, outsems = [in0, in1], [ad0, ad1], [ot0, ot1]

        pltpu.sync_copy(pos_hbm, pos_v)

        iot = lax.iota(jnp.int32, _LANES)
        for s in range(_NSLOT):
            rowbase = tile_base + s * maxlen
            for o in (0, 16, 32, 48, 64, 80, _CH[0] - _LANES):
                idxa[s, pl.ds(o, _LANES)] = rowbase + o + iot
            for o in (0, 16, 32, 48, 64, _CH[1] - _LANES):
                idxb[s, pl.ds(o, _LANES)] = rowbase + _CH[0] + o + iot

        def sp_slot(s):
            return sp.at[pl.ds(tile_base + s * maxlen, maxlen)]

        def start_in(r, s):
            pltpu.async_copy(x_hbm.at[base + r], sp_slot(s), insems[s])

        def wait_in(r, s):
            pltpu.make_async_copy(
                x_hbm.at[base + r], sp_slot(s), insems[s]).wait()

        def add_pos(s):
            pltpu.async_copy(pos_v.at[pl.ds(0, _CH[0])],
                             sp.at[idxa.at[s]], addsems[s], add=True)
            pltpu.async_copy(pos_v.at[pl.ds(_CH[0], _CH[1])],
                             sp.at[idxb.at[s]], addsems[s], add=True)
            pltpu.make_async_copy(pos_v.at[pl.ds(0, _CH[0])],
                                  sp.at[idxa.at[s]], addsems[s]).wait()
            pltpu.make_async_copy(pos_v.at[pl.ds(_CH[0], _CH[1])],
                                  sp.at[idxb.at[s]], addsems[s]).wait()

        def start_out(r, s):
            pltpu.async_copy(sp_slot(s), out_hbm.at[base + r], outsems[s])

        def wait_out(r, s):
            pltpu.make_async_copy(
                sp_slot(s), out_hbm.at[base + r], outsems[s]).wait()

        start_in(0, 0)
        start_in(1, 1)
        for r in range(b_per_w):
            s = r % _NSLOT
            wait_in(r, s)
            add_pos(s)
            start_out(r, s)
            if 1 <= r < b_per_w - 1:
                wait_out(r - 1, 1 - s)
                start_in(r + 1, 1 - s)
        wait_out(b_per_w - 2, (b_per_w - 2) % _NSLOT)
        wait_out(b_per_w - 1, (b_per_w - 1) % _NSLOT)

    return sc_add


def kernel(x, pos_emb):
    batch, maxlen, dim = x.shape
    return _make_sc_add(batch, maxlen, dim)(x, pos_emb)


# speedup vs baseline: 2.2094x; 1.0945x over previous
import functools

import jax
import jax.numpy as jnp
from jax import lax
from jax.experimental import pallas as pl
from jax.experimental.pallas import tpu as pltpu
from jax.experimental.pallas import tpu_sc as plsc

_LANES = 16
_NSLOT = 3
_CH = (104, 96)


def _make_sc_add(batch, maxlen, dim):
    info = plsc.get_sparse_core_info()
    nc, ns = info.num_cores, info.num_subcores
    nw = nc * ns
    assert batch % nw == 0 and dim % _LANES == 0 and sum(_CH) == maxlen
    b_per_w = batch // nw
    sp_rows = ns * _NSLOT * maxlen

    mesh = plsc.VectorSubcoreMesh(core_axis_name="c", subcore_axis_name="s")

    @functools.partial(
        pl.kernel,
        out_type=jax.ShapeDtypeStruct((batch, maxlen, dim), jnp.float32),
        mesh=mesh,
        scratch_types=[
            pltpu.VMEM((maxlen, dim), jnp.float32),
            pltpu.VMEM((_NSLOT, _CH[0]), jnp.int32),
            pltpu.VMEM((_NSLOT, _CH[1]), jnp.int32),
            pltpu.VMEM_SHARED((sp_rows, dim), jnp.float32),
            pltpu.SemaphoreType.DMA,
            pltpu.SemaphoreType.DMA,
            pltpu.SemaphoreType.DMA,
            pltpu.SemaphoreType.DMA,
            pltpu.SemaphoreType.DMA,
            pltpu.SemaphoreType.DMA,
            pltpu.SemaphoreType.DMA,
            pltpu.SemaphoreType.DMA,
            pltpu.SemaphoreType.DMA,
        ],
    )
    def sc_add(x_hbm, pos_hbm, out_hbm, pos_v, idxa, idxb, sp,
               in0, in1, in2, ad0, ad1, ad2, ot0, ot1, ot2):
        cid = lax.axis_index("c")
        sid = lax.axis_index("s")
        wid = sid * nc + cid
        base = wid * b_per_w
        tile_base = sid * (_NSLOT * maxlen)
        insems, addsems, outsems = [in0, in1, in2], [ad0, ad1, ad2], [ot0, ot1, ot2]

        pltpu.sync_copy(pos_hbm, pos_v)

        iot = lax.iota(jnp.int32, _LANES)
        for s in range(_NSLOT):
            rowbase = tile_base + s * maxlen
            for o in (0, 16, 32, 48, 64, 80, _CH[0] - _LANES):
                idxa[s, pl.ds(o, _LANES)] = rowbase + o + iot
            for o in (0, 16, 32, 48, 64, _CH[1] - _LANES):
                idxb[s, pl.ds(o, _LANES)] = rowbase + _CH[0] + o + iot

        def sp_slot(s):
            return sp.at[pl.ds(tile_base + s * maxlen, maxlen)]

        def start_in(r, s):
            pltpu.async_copy(x_hbm.at[base + r], sp_slot(s), insems[s])

        def wait_in(r, s):
            pltpu.make_async_copy(
                x_hbm.at[base + r], sp_slot(s), insems[s]).wait()

        def start_adds(s):
            pltpu.async_copy(pos_v.at[pl.ds(0, _CH[0])],
                             sp.at[idxa.at[s]], addsems[s], add=True)
            pltpu.async_copy(pos_v.at[pl.ds(_CH[0], _CH[1])],
                             sp.at[idxb.at[s]], addsems[s], add=True)

        def wait_adds(s):
            pltpu.make_async_copy(pos_v.at[pl.ds(0, _CH[0])],
                                  sp.at[idxa.at[s]], addsems[s]).wait()
            pltpu.make_async_copy(pos_v.at[pl.ds(_CH[0], _CH[1])],
                                  sp.at[idxb.at[s]], addsems[s]).wait()

        def start_out(r, s):
            pltpu.async_copy(sp_slot(s), out_hbm.at[base + r], outsems[s])

        def wait_out(r, s):
            pltpu.make_async_copy(
                sp_slot(s), out_hbm.at[base + r], outsems[s]).wait()

        for s in range(_NSLOT):
            start_in(s, s)
        for r in range(b_per_w):
            s = r % _NSLOT
            wait_in(r, s)
            start_adds(s)
            if r >= 1:
                q = (r - 1) % _NSLOT
                wait_adds(q)
                start_out(r - 1, q)
            if 2 <= r < b_per_w - 1:
                t = (r + 1) % _NSLOT
                wait_out(r - 2, t)
                start_in(r + 1, t)
        last = b_per_w - 1
        wait_adds(last % _NSLOT)
        start_out(last, last % _NSLOT)
        wait_out(last - 1, (last - 1) % _NSLOT)
        wait_out(last, last % _NSLOT)

    return sc_add


def kernel(x, pos_emb):
    batch, maxlen, dim = x.shape
    return _make_sc_add(batch, maxlen, dim)(x, pos_emb)
